# Initial kernel scaffold; baseline (speedup 1.0000x reference)
import jax
import jax.numpy as jnp
from jax.experimental import pallas as pl


def kernel(x, edge_index, W1, b1, W2, b2, W3, b3, ln1_w, ln1_b, ln2_w, ln2_b, ln3_w, ln3_b):
    raise NotImplementedError("write your pallas kernel here")


# speedup vs baseline: 4.0257x; 4.0257x over previous
import functools

import jax
import jax.numpy as jnp
from jax import lax
from jax.experimental import pallas as pl
from jax.experimental.pallas import tpu as pltpu
from jax.experimental.pallas import tpu_sc as plsc

N = 10000
IN_DIM = 256
HID = 512
EPS = 1e-5
NC, NS = 2, 16
NW = NC * NS
E_PAD = 163840
EPW = E_PAD // NW
NB = EPW // 128
ACC = 10112
RPW = ACC // NS
RB = 400
NRB = N // RB


def _sc_mesh():
    return plsc.VectorSubcoreMesh(core_axis_name="c", subcore_axis_name="s",
                                  num_cores=NC, num_subcores=NS)


def _sc_degree(col, zeros_c, ones_b):
    @functools.partial(
        pl.kernel,
        out_type=jax.ShapeDtypeStruct((NC, ACC, 128), jnp.float32),
        mesh=_sc_mesh(),
        scratch_types=[
            pltpu.VMEM((128,), jnp.int32),
            pltpu.VMEM((128, 128), jnp.float32),
            pltpu.VMEM_SHARED((ACC, 128), jnp.float32),
        ],
    )
    def k(col_h, zeros_h, ones_h, out_h, idx_v, ones_v, acc_sh):
        c = lax.axis_index("c")
        s = lax.axis_index("s")
        base = (c * NS + s) * EPW
        pltpu.sync_copy(ones_h, ones_v)
        pltpu.sync_copy(zeros_h, acc_sh.at[pl.ds(s * RPW, RPW)])
        plsc.subcore_barrier()

        @pl.loop(0, NB)
        def _(b):
            pltpu.sync_copy(col_h.at[pl.ds(base + b * 128, 128)], idx_v)
            pltpu.sync_copy(ones_v, acc_sh.at[idx_v], add=True)

        plsc.subcore_barrier()
        pltpu.sync_copy(acc_sh.at[pl.ds(s * RPW, RPW)],
                        out_h.at[c, pl.ds(s * RPW, RPW)])

    return k(col, zeros_c, ones_b)


def _sc_aggregate(hs, row, col, zeros_c):
    P, _, C = hs.shape

    @functools.partial(
        pl.kernel,
        out_type=jax.ShapeDtypeStruct((NC, P, ACC, C), jnp.float32),
        mesh=_sc_mesh(),
        scratch_types=[
            pltpu.VMEM((128,), jnp.int32),
            pltpu.VMEM((128,), jnp.int32),
            pltpu.VMEM((128, C), jnp.float32),
            pltpu.VMEM_SHARED((ACC, C), jnp.float32),
        ],
    )
    def k(hs_h, row_h, col_h, zeros_h, out_h, row_v, col_v, buf_v, acc_sh):
        c = lax.axis_index("c")
        s = lax.axis_index("s")
        base = (c * NS + s) * EPW
        for p in range(P):
            pltpu.sync_copy(zeros_h, acc_sh.at[pl.ds(s * RPW, RPW)])
            plsc.subcore_barrier()

            @pl.loop(0, NB)
            def _(b):
                off = base + b * 128
                pltpu.sync_copy(row_h.at[pl.ds(off, 128)], row_v)
                pltpu.sync_copy(col_h.at[pl.ds(off, 128)], col_v)
                pltpu.sync_copy(hs_h.at[p].at[row_v], buf_v)
                pltpu.sync_copy(buf_v, acc_sh.at[col_v], add=True)

            plsc.subcore_barrier()
            pltpu.sync_copy(acc_sh.at[pl.ds(s * RPW, RPW)],
                            out_h.at[c, p, pl.ds(s * RPW, RPW)])
            if p + 1 < P:
                plsc.subcore_barrier()

    return k(hs, row, col, zeros_c)


def _tc_prescale(degp, x):
    def body(deg_ref, x_ref, dinv_ref, hs_ref):
        d = deg_ref[0] + deg_ref[1] + 1.0
        dv = lax.rsqrt(d)
        dinv_ref[...] = dv[:, 0:16]
        hs_ref[0] = dv[:, 0:1] * x_ref[...]

    return pl.pallas_call(
        body,
        grid=(NRB, IN_DIM // 128),
        in_specs=[
            pl.BlockSpec((NC, RB, 128), lambda i, p: (0, i, 0)),
            pl.BlockSpec((RB, 128), lambda i, p: (i, p)),
        ],
        out_specs=[
            pl.BlockSpec((RB, 16), lambda i, p: (i, 0)),
            pl.BlockSpec((1, RB, 128), lambda i, p: (p, i, 0)),
        ],
        out_shape=[
            jax.ShapeDtypeStruct((N, 16), jnp.float32),
            jax.ShapeDtypeStruct((IN_DIM // 128, N, 128), jnp.float32),
        ],
    )(degp, x)


def _tc_agg_matmul(partials, hs, dinv16, W, bias):
    P = hs.shape[0]
    H = W.shape[1]

    def body(part_ref, hs_ref, dinv_ref, w_ref, b_ref, h_ref, st_ref):
        i = pl.program_id(0)
        dv = dinv_ref[...][:, 0:1]
        z = jnp.zeros((RB, H), jnp.float32)
        for p in range(P):
            aggp = dv * (part_ref[0, p] + part_ref[1, p] + hs_ref[p])
            z = z + jnp.dot(aggp, w_ref[pl.ds(p * 128, 128), :],
                            preferred_element_type=jnp.float32)
        h = jnp.maximum(z + b_ref[...], 0.0)
        h_ref[...] = h
        s = jnp.sum(h)
        s2 = jnp.sum(h * h)
        vec = jnp.concatenate([jnp.full((1, 128), s, jnp.float32),
                               jnp.full((1, 128), s2, jnp.float32)], axis=1)

        @pl.when(i == 0)
        def _():
            st_ref[...] = jnp.zeros((1, 256), jnp.float32)

        st_ref[...] += vec

    return pl.pallas_call(
        body,
        grid=(NRB,),
        in_specs=[
            pl.BlockSpec((NC, P, RB, 128), lambda i: (0, 0, i, 0)),
            pl.BlockSpec((P, RB, 128), lambda i: (0, i, 0)),
            pl.BlockSpec((RB, 16), lambda i: (i, 0)),
            pl.BlockSpec((P * 128, H), lambda i: (0, 0)),
            pl.BlockSpec((1, H), lambda i: (0, 0)),
        ],
        out_specs=[
            pl.BlockSpec((RB, H), lambda i: (i, 0)),
            pl.BlockSpec((1, 256), lambda i: (0, 0)),
        ],
        out_shape=[
            jax.ShapeDtypeStruct((N, H), jnp.float32),
            jax.ShapeDtypeStruct((1, 256), jnp.float32),
        ],
    )(partials, hs, dinv16, W, bias)


def _tc_norm_prescale(h, stats, lnw, lnb, dinv16):
    nelem = float(N * HID)

    def body(h_ref, st_ref, w_ref, b_ref, dinv_ref, out_ref):
        mu = st_ref[0, 0] / nelem
        ms = st_ref[0, 128] / nelem
        inv = 1.0 / (jnp.sqrt(jnp.maximum(ms - mu * mu, 0.0)) + EPS)
        dv = dinv_ref[...][:, 0:1]
        hn = (h_ref[...] - mu) * inv * w_ref[...] + b_ref[...]
        out_ref[0] = dv * hn

    return pl.pallas_call(
        body,
        grid=(NRB, HID // 128),
        in_specs=[
            pl.BlockSpec((RB, 128), lambda i, p: (i, p)),
            pl.BlockSpec((1, 256), lambda i, p: (0, 0)),
            pl.BlockSpec((1, 128), lambda i, p: (0, p)),
            pl.BlockSpec((1, 128), lambda i, p: (0, p)),
            pl.BlockSpec((RB, 16), lambda i, p: (i, 0)),
        ],
        out_specs=pl.BlockSpec((1, RB, 128), lambda i, p: (p, i, 0)),
        out_shape=jax.ShapeDtypeStruct((HID // 128, N, 128), jnp.float32),
    )(h, stats, lnw, lnb, dinv16)


def _tc_norm_matmul3(h, stats, lnw, lnb, dinv16, w3t):
    nelem = float(N * HID)

    def body(h_ref, st_ref, w_ref, b_ref, dinv_ref, w3_ref, out_ref):
        mu = st_ref[0, 0] / nelem
        ms = st_ref[0, 128] / nelem
        inv = 1.0 / (jnp.sqrt(jnp.maximum(ms - mu * mu, 0.0)) + EPS)
        hn = (h_ref[...] - mu) * inv * w_ref[...] + b_ref[...]
        t = jnp.dot(hn, w3_ref[...], preferred_element_type=jnp.float32)
        out_ref[...] = dinv_ref[...][:, 0:1] * t

    return pl.pallas_call(
        body,
        grid=(NRB,),
        in_specs=[
            pl.BlockSpec((RB, HID), lambda i: (i, 0)),
            pl.BlockSpec((1, 256), lambda i: (0, 0)),
            pl.BlockSpec((1, HID), lambda i: (0, 0)),
            pl.BlockSpec((1, HID), lambda i: (0, 0)),
            pl.BlockSpec((RB, 16), lambda i: (i, 0)),
            pl.BlockSpec((HID, 128), lambda i: (0, 0)),
        ],
        out_specs=pl.BlockSpec((RB, 128), lambda i: (i, 0)),
        out_shape=jax.ShapeDtypeStruct((N, 128), jnp.float32),
    )(h, stats, lnw, lnb, dinv16, w3t)


def _tc_final(p3, ts, dinv16, b3t, lnw3t, lnb3t):
    def body(p_ref, ts_ref, dinv_ref, b3_ref, w_ref, b_ref, out_ref):
        psum = (p_ref[0, 0] + p_ref[1, 0])[0:N, :]
        y = dinv_ref[...][:, 0:1] * (psum + ts_ref[...])
        h = jnp.maximum(y + b3_ref[...], 0.0)
        col = h[:, 0:1]
        mu = jnp.sum(col) / N
        ms = jnp.sum(col * col) / N
        inv = 1.0 / (jnp.sqrt(jnp.maximum(ms - mu * mu, 0.0)) + EPS)
        out_ref[...] = (h - mu) * inv * w_ref[...] + b_ref[...]

    return pl.pallas_call(
        body,
        in_specs=[
            pl.BlockSpec((NC, 1, ACC, 128), lambda: (0, 0, 0, 0)),
            pl.BlockSpec((N, 128), lambda: (0, 0)),
            pl.BlockSpec((N, 16), lambda: (0, 0)),
            pl.BlockSpec((1, 128), lambda: (0, 0)),
            pl.BlockSpec((1, 128), lambda: (0, 0)),
            pl.BlockSpec((1, 128), lambda: (0, 0)),
        ],
        out_specs=pl.BlockSpec((N, 128), lambda: (0, 0)),
        out_shape=jax.ShapeDtypeStruct((N, 128), jnp.float32),
    )(p3, ts, dinv16, b3t, lnw3t, lnb3t)


def kernel(x, edge_index, W1, b1, W2, b2, W3, b3,
           ln1_w, ln1_b, ln2_w, ln2_b, ln3_w, ln3_b):
    ei = edge_index.astype(jnp.int32)
    e = ei.shape[1]
    row = jnp.concatenate([ei[0], jnp.zeros((E_PAD - e,), jnp.int32)])
    col = jnp.concatenate([ei[1], jnp.full((E_PAD - e,), N, jnp.int32)])
    zeros128 = jnp.zeros((RPW, 128), jnp.float32)
    ones_b = jnp.ones((128, 128), jnp.float32)

    degp = _sc_degree(col, zeros128, ones_b)
    dinv16, hs0 = _tc_prescale(degp, x)

    p1 = _sc_aggregate(hs0, row, col, zeros128)
    h1, st1 = _tc_agg_matmul(p1, hs0, dinv16, W1, b1.reshape(1, -1))
    hs1 = _tc_norm_prescale(h1, st1, ln1_w.reshape(1, -1),
                            ln1_b.reshape(1, -1), dinv16)

    p2 = _sc_aggregate(hs1, row, col, zeros128)
    h2, st2 = _tc_agg_matmul(p2, hs1, dinv16, W2, b2.reshape(1, -1))
    ts = _tc_norm_matmul3(h2, st2, ln2_w.reshape(1, -1),
                          ln2_b.reshape(1, -1), dinv16,
                          jnp.tile(W3, (1, 128)))

    p3 = _sc_aggregate(ts.reshape(1, N, 128), row, col, zeros128)
    out128 = _tc_final(p3, ts, dinv16,
                       jnp.broadcast_to(b3.reshape(1, 1), (1, 128)),
                       jnp.broadcast_to(ln3_w.reshape(1, 1), (1, 128)),
                       jnp.broadcast_to(ln3_b.reshape(1, 1), (1, 128)))
    return out128[:, 0]


# speedup vs baseline: 4.9084x; 1.2193x over previous
import functools

import jax
import jax.numpy as jnp
from jax import lax
from jax.experimental import pallas as pl
from jax.experimental.pallas import tpu as pltpu
from jax.experimental.pallas import tpu_sc as plsc

N = 10000
IN_DIM = 256
HID = 512
EPS = 1e-5
NC, NS = 2, 16
NW = NC * NS
E_PAD = 163840
EPW = E_PAD // NW
NB = EPW // 128
ACC = 10112
RPW = ACC // NS
RB = 400
NRB = N // RB


def _sc_mesh():
    return plsc.VectorSubcoreMesh(core_axis_name="c", subcore_axis_name="s",
                                  num_cores=NC, num_subcores=NS)


def _sc_degree(col, zeros_c, ones_b):
    @functools.partial(
        pl.kernel,
        out_type=jax.ShapeDtypeStruct((NC, ACC, 128), jnp.float32),
        mesh=_sc_mesh(),
        scratch_types=[
            pltpu.VMEM((128,), jnp.int32),
            pltpu.VMEM((128, 128), jnp.float32),
            pltpu.VMEM_SHARED((ACC, 128), jnp.float32),
        ],
    )
    def k(col_h, zeros_h, ones_h, out_h, idx_v, ones_v, acc_sh):
        c = lax.axis_index("c")
        s = lax.axis_index("s")
        base = (c * NS + s) * EPW
        pltpu.sync_copy(ones_h, ones_v)
        pltpu.sync_copy(zeros_h, acc_sh.at[pl.ds(s * RPW, RPW)])
        plsc.subcore_barrier()

        @pl.loop(0, NB)
        def _(b):
            pltpu.sync_copy(col_h.at[pl.ds(base + b * 128, 128)], idx_v)
            pltpu.sync_copy(ones_v, acc_sh.at[idx_v], add=True)

        plsc.subcore_barrier()
        pltpu.sync_copy(acc_sh.at[pl.ds(s * RPW, RPW)],
                        out_h.at[c, pl.ds(s * RPW, RPW)])

    return k(col, zeros_c, ones_b)


def _sc_aggregate(hs, row, col, zeros_c):
    P, _, C = hs.shape

    @functools.partial(
        pl.kernel,
        out_type=jax.ShapeDtypeStruct((NC, P, ACC, C), jnp.float32),
        mesh=_sc_mesh(),
        scratch_types=[
            pltpu.VMEM((2, 128), jnp.int32),
            pltpu.VMEM((2, 128), jnp.int32),
            pltpu.VMEM((2, 128, C), jnp.float32),
            pltpu.VMEM_SHARED((ACC, C), jnp.float32),
            pltpu.SemaphoreType.DMA,
            pltpu.SemaphoreType.DMA,
        ],
    )
    def k(hs_h, row_h, col_h, zeros_h, out_h, row_v, col_v, buf_v, acc_sh,
          sem0, sem1):
        c = lax.axis_index("c")
        s = lax.axis_index("s")
        base = (c * NS + s) * EPW
        sems = (sem0, sem1)

        def load_idx(b, j):
            pltpu.sync_copy(row_h.at[pl.ds(base + b * 128, 128)], row_v.at[j])
            pltpu.sync_copy(col_h.at[pl.ds(base + b * 128, 128)], col_v.at[j])

        def fire(p, j):
            pltpu.async_copy(hs_h.at[p].at[row_v.at[j]], buf_v.at[j], sems[j])

        def wait(p, j):
            pltpu.make_async_copy(hs_h.at[p].at[row_v.at[j]], buf_v.at[j],
                                  sems[j]).wait()

        def scat(j):
            pltpu.sync_copy(buf_v.at[j], acc_sh.at[col_v.at[j]], add=True)

        for p in range(P):
            pltpu.sync_copy(zeros_h, acc_sh.at[pl.ds(s * RPW, RPW)])
            plsc.subcore_barrier()

            load_idx(0, 0)
            fire(p, 0)

            @pl.loop(0, (NB - 2) // 2)
            def _(i):
                bb = i * 2
                for j in range(2):
                    load_idx(bb + j + 1, 1 - j)
                    fire(p, 1 - j)
                    wait(p, j)
                    scat(j)

            load_idx(NB - 1, 1)
            fire(p, 1)
            wait(p, 0)
            scat(0)
            wait(p, 1)
            scat(1)

            plsc.subcore_barrier()
            pltpu.sync_copy(acc_sh.at[pl.ds(s * RPW, RPW)],
                            out_h.at[c, p, pl.ds(s * RPW, RPW)])
            if p + 1 < P:
                plsc.subcore_barrier()

    return k(hs, row, col, zeros_c)


def _tc_prescale(degp, x):
    def body(deg_ref, x_ref, dinv_ref, hs_ref):
        d = deg_ref[0] + deg_ref[1] + 1.0
        dv = lax.rsqrt(d)
        dinv_ref[...] = dv[:, 0:16]
        hs_ref[0] = dv[:, 0:1] * x_ref[...]

    return pl.pallas_call(
        body,
        grid=(NRB, IN_DIM // 128),
        in_specs=[
            pl.BlockSpec((NC, RB, 128), lambda i, p: (0, i, 0)),
            pl.BlockSpec((RB, 128), lambda i, p: (i, p)),
        ],
        out_specs=[
            pl.BlockSpec((RB, 16), lambda i, p: (i, 0)),
            pl.BlockSpec((1, RB, 128), lambda i, p: (p, i, 0)),
        ],
        out_shape=[
            jax.ShapeDtypeStruct((N, 16), jnp.float32),
            jax.ShapeDtypeStruct((IN_DIM // 128, N, 128), jnp.float32),
        ],
    )(degp, x)


def _tc_agg_matmul(partials, hs, dinv16, W, bias):
    P = hs.shape[0]
    H = W.shape[1]

    def body(part_ref, hs_ref, dinv_ref, w_ref, b_ref, h_ref, st_ref):
        i = pl.program_id(0)
        dv = dinv_ref[...][:, 0:1]
        z = jnp.zeros((RB, H), jnp.float32)
        for p in range(P):
            aggp = dv * (part_ref[0, p] + part_ref[1, p] + hs_ref[p])
            z = z + jnp.dot(aggp, w_ref[pl.ds(p * 128, 128), :],
                            preferred_element_type=jnp.float32)
        h = jnp.maximum(z + b_ref[...], 0.0)
        h_ref[...] = h
        s = jnp.sum(h)
        s2 = jnp.sum(h * h)
        vec = jnp.concatenate([jnp.full((1, 128), s, jnp.float32),
                               jnp.full((1, 128), s2, jnp.float32)], axis=1)

        @pl.when(i == 0)
        def _():
            st_ref[...] = jnp.zeros((1, 256), jnp.float32)

        st_ref[...] += vec

    return pl.pallas_call(
        body,
        grid=(NRB,),
        in_specs=[
            pl.BlockSpec((NC, P, RB, 128), lambda i: (0, 0, i, 0)),
            pl.BlockSpec((P, RB, 128), lambda i: (0, i, 0)),
            pl.BlockSpec((RB, 16), lambda i: (i, 0)),
            pl.BlockSpec((P * 128, H), lambda i: (0, 0)),
            pl.BlockSpec((1, H), lambda i: (0, 0)),
        ],
        out_specs=[
            pl.BlockSpec((RB, H), lambda i: (i, 0)),
            pl.BlockSpec((1, 256), lambda i: (0, 0)),
        ],
        out_shape=[
            jax.ShapeDtypeStruct((N, H), jnp.float32),
            jax.ShapeDtypeStruct((1, 256), jnp.float32),
        ],
    )(partials, hs, dinv16, W, bias)


def _tc_norm_prescale(h, stats, lnw, lnb, dinv16):
    nelem = float(N * HID)

    def body(h_ref, st_ref, w_ref, b_ref, dinv_ref, out_ref):
        mu = st_ref[0, 0] / nelem
        ms = st_ref[0, 128] / nelem
        inv = 1.0 / (jnp.sqrt(jnp.maximum(ms - mu * mu, 0.0)) + EPS)
        dv = dinv_ref[...][:, 0:1]
        hn = (h_ref[...] - mu) * inv * w_ref[...] + b_ref[...]
        out_ref[0] = dv * hn

    return pl.pallas_call(
        body,
        grid=(NRB, HID // 128),
        in_specs=[
            pl.BlockSpec((RB, 128), lambda i, p: (i, p)),
            pl.BlockSpec((1, 256), lambda i, p: (0, 0)),
            pl.BlockSpec((1, 128), lambda i, p: (0, p)),
            pl.BlockSpec((1, 128), lambda i, p: (0, p)),
            pl.BlockSpec((RB, 16), lambda i, p: (i, 0)),
        ],
        out_specs=pl.BlockSpec((1, RB, 128), lambda i, p: (p, i, 0)),
        out_shape=jax.ShapeDtypeStruct((HID // 128, N, 128), jnp.float32),
    )(h, stats, lnw, lnb, dinv16)


def _tc_norm_matmul3(h, stats, lnw, lnb, dinv16, w3t):
    nelem = float(N * HID)

    def body(h_ref, st_ref, w_ref, b_ref, dinv_ref, w3_ref, out_ref):
        mu = st_ref[0, 0] / nelem
        ms = st_ref[0, 128] / nelem
        inv = 1.0 / (jnp.sqrt(jnp.maximum(ms - mu * mu, 0.0)) + EPS)
        hn = (h_ref[...] - mu) * inv * w_ref[...] + b_ref[...]
        t = jnp.dot(hn, w3_ref[...], preferred_element_type=jnp.float32)
        out_ref[...] = dinv_ref[...][:, 0:1] * t

    return pl.pallas_call(
        body,
        grid=(NRB,),
        in_specs=[
            pl.BlockSpec((RB, HID), lambda i: (i, 0)),
            pl.BlockSpec((1, 256), lambda i: (0, 0)),
            pl.BlockSpec((1, HID), lambda i: (0, 0)),
            pl.BlockSpec((1, HID), lambda i: (0, 0)),
            pl.BlockSpec((RB, 16), lambda i: (i, 0)),
            pl.BlockSpec((HID, 128), lambda i: (0, 0)),
        ],
        out_specs=pl.BlockSpec((RB, 128), lambda i: (i, 0)),
        out_shape=jax.ShapeDtypeStruct((N, 128), jnp.float32),
    )(h, stats, lnw, lnb, dinv16, w3t)


def _tc_final(p3, ts, dinv16, b3t, lnw3t, lnb3t):
    def body(p_ref, ts_ref, dinv_ref, b3_ref, w_ref, b_ref, out_ref):
        psum = (p_ref[0, 0] + p_ref[1, 0])[0:N, :]
        y = dinv_ref[...][:, 0:1] * (psum + ts_ref[...])
        h = jnp.maximum(y + b3_ref[...], 0.0)
        col = h[:, 0:1]
        mu = jnp.sum(col) / N
        ms = jnp.sum(col * col) / N
        inv = 1.0 / (jnp.sqrt(jnp.maximum(ms - mu * mu, 0.0)) + EPS)
        out_ref[...] = (h - mu) * inv * w_ref[...] + b_ref[...]

    return pl.pallas_call(
        body,
        in_specs=[
            pl.BlockSpec((NC, 1, ACC, 128), lambda: (0, 0, 0, 0)),
            pl.BlockSpec((N, 128), lambda: (0, 0)),
            pl.BlockSpec((N, 16), lambda: (0, 0)),
            pl.BlockSpec((1, 128), lambda: (0, 0)),
            pl.BlockSpec((1, 128), lambda: (0, 0)),
            pl.BlockSpec((1, 128), lambda: (0, 0)),
        ],
        out_specs=pl.BlockSpec((N, 128), lambda: (0, 0)),
        out_shape=jax.ShapeDtypeStruct((N, 128), jnp.float32),
    )(p3, ts, dinv16, b3t, lnw3t, lnb3t)


def kernel(x, edge_index, W1, b1, W2, b2, W3, b3,
           ln1_w, ln1_b, ln2_w, ln2_b, ln3_w, ln3_b):
    ei = edge_index.astype(jnp.int32)
    e = ei.shape[1]
    row = jnp.concatenate([ei[0], jnp.zeros((E_PAD - e,), jnp.int32)])
    col = jnp.concatenate([ei[1], jnp.full((E_PAD - e,), N, jnp.int32)])
    zeros128 = jnp.zeros((RPW, 128), jnp.float32)
    ones_b = jnp.ones((128, 128), jnp.float32)

    degp = _sc_degree(col, zeros128, ones_b)
    dinv16, hs0 = _tc_prescale(degp, x)

    p1 = _sc_aggregate(hs0, row, col, zeros128)
    h1, st1 = _tc_agg_matmul(p1, hs0, dinv16, W1, b1.reshape(1, -1))
    hs1 = _tc_norm_prescale(h1, st1, ln1_w.reshape(1, -1),
                            ln1_b.reshape(1, -1), dinv16)

    p2 = _sc_aggregate(hs1, row, col, zeros128)
    h2, st2 = _tc_agg_matmul(p2, hs1, dinv16, W2, b2.reshape(1, -1))
    ts = _tc_norm_matmul3(h2, st2, ln2_w.reshape(1, -1),
                          ln2_b.reshape(1, -1), dinv16,
                          jnp.tile(W3, (1, 128)))

    p3 = _sc_aggregate(ts.reshape(1, N, 128), row, col, zeros128)
    out128 = _tc_final(p3, ts, dinv16,
                       jnp.broadcast_to(b3.reshape(1, 1), (1, 128)),
                       jnp.broadcast_to(ln3_w.reshape(1, 1), (1, 128)),
                       jnp.broadcast_to(ln3_b.reshape(1, 1), (1, 128)))
    return out128[:, 0]


# speedup vs baseline: 5.1857x; 1.0565x over previous
import functools

import jax
import jax.numpy as jnp
from jax import lax
from jax.experimental import pallas as pl
from jax.experimental.pallas import tpu as pltpu
from jax.experimental.pallas import tpu_sc as plsc

N = 10000
IN_DIM = 256
HID = 512
EPS = 1e-5
NC, NS = 2, 16
NW = NC * NS
E_PAD = 163840
EPW = E_PAD // NW
NB = EPW // 128
ACC = 10112
RPW = ACC // NS
RB = 400
NRB = N // RB


def _sc_mesh():
    return plsc.VectorSubcoreMesh(core_axis_name="c", subcore_axis_name="s",
                                  num_cores=NC, num_subcores=NS)


def _sc_degree(col, zeros_c, ones_b):
    @functools.partial(
        pl.kernel,
        out_type=jax.ShapeDtypeStruct((NC, ACC, 128), jnp.float32),
        mesh=_sc_mesh(),
        scratch_types=[
            pltpu.VMEM((128,), jnp.int32),
            pltpu.VMEM((128, 128), jnp.float32),
            pltpu.VMEM_SHARED((ACC, 128), jnp.float32),
        ],
    )
    def k(col_h, zeros_h, ones_h, out_h, idx_v, ones_v, acc_sh):
        c = lax.axis_index("c")
        s = lax.axis_index("s")
        wb = (c * NS + s) * NB
        pltpu.sync_copy(ones_h, ones_v)
        pltpu.sync_copy(zeros_h, acc_sh.at[pl.ds(s * RPW, RPW)])
        plsc.subcore_barrier()

        @pl.loop(0, NB)
        def _(b):
            pltpu.sync_copy(col_h.at[wb + b], idx_v)
            pltpu.sync_copy(ones_v, acc_sh.at[idx_v], add=True)

        plsc.subcore_barrier()
        pltpu.sync_copy(acc_sh.at[pl.ds(s * RPW, RPW)],
                        out_h.at[c, pl.ds(s * RPW, RPW)])

    return k(col, zeros_c, ones_b)


def _sc_aggregate(hs, row, col, zeros_c):
    P, _, C = hs.shape

    @functools.partial(
        pl.kernel,
        out_type=jax.ShapeDtypeStruct((NC, P, ACC, C), jnp.float32),
        mesh=_sc_mesh(),
        scratch_types=[
            pltpu.VMEM((NB, 128), jnp.int32),
            pltpu.VMEM((NB, 128), jnp.int32),
            pltpu.VMEM((2, 128, C), jnp.float32),
            pltpu.VMEM_SHARED((ACC, C), jnp.float32),
            pltpu.SemaphoreType.DMA,
            pltpu.SemaphoreType.DMA,
            pltpu.SemaphoreType.DMA,
            pltpu.SemaphoreType.DMA,
        ],
    )
    def k(hs_h, row_h, col_h, zeros_h, out_h, row_i, col_i, buf_v, acc_sh,
          g0, g1, s0, s1):
        c = lax.axis_index("c")
        s = lax.axis_index("s")
        wb = (c * NS + s) * NB
        gsem = (g0, g1)
        ssem = (s0, s1)

        pltpu.sync_copy(row_h.at[pl.ds(wb, NB)], row_i)
        pltpu.sync_copy(col_h.at[pl.ds(wb, NB)], col_i)

        def fire_g(p, b, j):
            pltpu.async_copy(hs_h.at[p].at[row_i.at[b]], buf_v.at[j], gsem[j])

        def wait_g(p, b, j):
            pltpu.make_async_copy(hs_h.at[p].at[row_i.at[b]], buf_v.at[j],
                                  gsem[j]).wait()

        def fire_s(b, j):
            pltpu.async_copy(buf_v.at[j], acc_sh.at[col_i.at[b]], ssem[j],
                             add=True)

        def wait_s(b, j):
            pltpu.make_async_copy(buf_v.at[j], acc_sh.at[col_i.at[b]],
                                  ssem[j]).wait()

        for p in range(P):
            pltpu.sync_copy(zeros_h, acc_sh.at[pl.ds(s * RPW, RPW)])
            plsc.subcore_barrier()

            fire_g(p, 0, 0)
            fire_g(p, 1, 1)
            wait_g(p, 0, 0)
            fire_s(0, 0)

            @pl.loop(0, (NB - 2) // 2)
            def _(i):
                for j in range(2):
                    b = 1 + i * 2 + j
                    cur = 1 - j
                    nxt = j
                    wait_s(b - 1, nxt)
                    fire_g(p, b + 1, nxt)
                    wait_g(p, b, cur)
                    fire_s(b, cur)

            wait_g(p, NB - 1, 1)
            fire_s(NB - 1, 1)
            wait_s(NB - 2, 0)
            wait_s(NB - 1, 1)

            plsc.subcore_barrier()
            pltpu.sync_copy(acc_sh.at[pl.ds(s * RPW, RPW)],
                            out_h.at[c, p, pl.ds(s * RPW, RPW)])
            if p + 1 < P:
                plsc.subcore_barrier()

    return k(hs, row, col, zeros_c)


def _tc_prescale(degp, x):
    def body(deg_ref, x_ref, dinv_ref, hs_ref):
        d = deg_ref[0] + deg_ref[1] + 1.0
        dv = lax.rsqrt(d)
        dinv_ref[...] = dv[:, 0:16]
        hs_ref[0] = dv[:, 0:1] * x_ref[...]

    return pl.pallas_call(
        body,
        grid=(NRB, IN_DIM // 128),
        in_specs=[
            pl.BlockSpec((NC, RB, 128), lambda i, p: (0, i, 0)),
            pl.BlockSpec((RB, 128), lambda i, p: (i, p)),
        ],
        out_specs=[
            pl.BlockSpec((RB, 16), lambda i, p: (i, 0)),
            pl.BlockSpec((1, RB, 128), lambda i, p: (p, i, 0)),
        ],
        out_shape=[
            jax.ShapeDtypeStruct((N, 16), jnp.float32),
            jax.ShapeDtypeStruct((IN_DIM // 128, N, 128), jnp.float32),
        ],
    )(degp, x)


def _tc_agg_matmul(partials, hs, dinv16, W, bias):
    P = hs.shape[0]
    H = W.shape[1]

    def body(part_ref, hs_ref, dinv_ref, w_ref, b_ref, h_ref, st_ref):
        i = pl.program_id(0)
        dv = dinv_ref[...][:, 0:1]
        z = jnp.zeros((RB, H), jnp.float32)
        for p in range(P):
            aggp = dv * (part_ref[0, p] + part_ref[1, p] + hs_ref[p])
            z = z + jnp.dot(aggp, w_ref[pl.ds(p * 128, 128), :],
                            preferred_element_type=jnp.float32)
        h = jnp.maximum(z + b_ref[...], 0.0)
        h_ref[...] = h
        s = jnp.sum(h)
        s2 = jnp.sum(h * h)
        vec = jnp.concatenate([jnp.full((1, 128), s, jnp.float32),
                               jnp.full((1, 128), s2, jnp.float32)], axis=1)

        @pl.when(i == 0)
        def _():
            st_ref[...] = jnp.zeros((1, 256), jnp.float32)

        st_ref[...] += vec

    return pl.pallas_call(
        body,
        grid=(NRB,),
        in_specs=[
            pl.BlockSpec((NC, P, RB, 128), lambda i: (0, 0, i, 0)),
            pl.BlockSpec((P, RB, 128), lambda i: (0, i, 0)),
            pl.BlockSpec((RB, 16), lambda i: (i, 0)),
            pl.BlockSpec((P * 128, H), lambda i: (0, 0)),
            pl.BlockSpec((1, H), lambda i: (0, 0)),
        ],
        out_specs=[
            pl.BlockSpec((RB, H), lambda i: (i, 0)),
            pl.BlockSpec((1, 256), lambda i: (0, 0)),
        ],
        out_shape=[
            jax.ShapeDtypeStruct((N, H), jnp.float32),
            jax.ShapeDtypeStruct((1, 256), jnp.float32),
        ],
    )(partials, hs, dinv16, W, bias)


def _tc_norm_prescale(h, stats, lnw, lnb, dinv16):
    nelem = float(N * HID)

    def body(h_ref, st_ref, w_ref, b_ref, dinv_ref, out_ref):
        mu = st_ref[0, 0] / nelem
        ms = st_ref[0, 128] / nelem
        inv = 1.0 / (jnp.sqrt(jnp.maximum(ms - mu * mu, 0.0)) + EPS)
        dv = dinv_ref[...][:, 0:1]
        hn = (h_ref[...] - mu) * inv * w_ref[...] + b_ref[...]
        out_ref[0] = dv * hn

    return pl.pallas_call(
        body,
        grid=(NRB, HID // 128),
        in_specs=[
            pl.BlockSpec((RB, 128), lambda i, p: (i, p)),
            pl.BlockSpec((1, 256), lambda i, p: (0, 0)),
            pl.BlockSpec((1, 128), lambda i, p: (0, p)),
            pl.BlockSpec((1, 128), lambda i, p: (0, p)),
            pl.BlockSpec((RB, 16), lambda i, p: (i, 0)),
        ],
        out_specs=pl.BlockSpec((1, RB, 128), lambda i, p: (p, i, 0)),
        out_shape=jax.ShapeDtypeStruct((HID // 128, N, 128), jnp.float32),
    )(h, stats, lnw, lnb, dinv16)


def _tc_norm_matmul3(h, stats, lnw, lnb, dinv16, w3t):
    nelem = float(N * HID)

    def body(h_ref, st_ref, w_ref, b_ref, dinv_ref, w3_ref, out_ref):
        mu = st_ref[0, 0] / nelem
        ms = st_ref[0, 128] / nelem
        inv = 1.0 / (jnp.sqrt(jnp.maximum(ms - mu * mu, 0.0)) + EPS)
        hn = (h_ref[...] - mu) * inv * w_ref[...] + b_ref[...]
        t = jnp.dot(hn, w3_ref[...], preferred_element_type=jnp.float32)
        out_ref[...] = dinv_ref[...][:, 0:1] * t

    return pl.pallas_call(
        body,
        grid=(NRB,),
        in_specs=[
            pl.BlockSpec((RB, HID), lambda i: (i, 0)),
            pl.BlockSpec((1, 256), lambda i: (0, 0)),
            pl.BlockSpec((1, HID), lambda i: (0, 0)),
            pl.BlockSpec((1, HID), lambda i: (0, 0)),
            pl.BlockSpec((RB, 16), lambda i: (i, 0)),
            pl.BlockSpec((HID, 128), lambda i: (0, 0)),
        ],
        out_specs=pl.BlockSpec((RB, 128), lambda i: (i, 0)),
        out_shape=jax.ShapeDtypeStruct((N, 128), jnp.float32),
    )(h, stats, lnw, lnb, dinv16, w3t)


def _tc_final(p3, ts, dinv16, b3t, lnw3t, lnb3t):
    def body(p_ref, ts_ref, dinv_ref, b3_ref, w_ref, b_ref, out_ref):
        psum = (p_ref[0, 0] + p_ref[1, 0])[0:N, :]
        y = dinv_ref[...][:, 0:1] * (psum + ts_ref[...])
        h = jnp.maximum(y + b3_ref[...], 0.0)
        col = h[:, 0:1]
        mu = jnp.sum(col) / N
        ms = jnp.sum(col * col) / N
        inv = 1.0 / (jnp.sqrt(jnp.maximum(ms - mu * mu, 0.0)) + EPS)
        out_ref[...] = (h - mu) * inv * w_ref[...] + b_ref[...]

    return pl.pallas_call(
        body,
        in_specs=[
            pl.BlockSpec((NC, 1, ACC, 128), lambda: (0, 0, 0, 0)),
            pl.BlockSpec((N, 128), lambda: (0, 0)),
            pl.BlockSpec((N, 16), lambda: (0, 0)),
            pl.BlockSpec((1, 128), lambda: (0, 0)),
            pl.BlockSpec((1, 128), lambda: (0, 0)),
            pl.BlockSpec((1, 128), lambda: (0, 0)),
        ],
        out_specs=pl.BlockSpec((N, 128), lambda: (0, 0)),
        out_shape=jax.ShapeDtypeStruct((N, 128), jnp.float32),
    )(p3, ts, dinv16, b3t, lnw3t, lnb3t)


def kernel(x, edge_index, W1, b1, W2, b2, W3, b3,
           ln1_w, ln1_b, ln2_w, ln2_b, ln3_w, ln3_b):
    ei = edge_index.astype(jnp.int32)
    e = ei.shape[1]
    row = jnp.concatenate([ei[0], jnp.zeros((E_PAD - e,), jnp.int32)])
    col = jnp.concatenate([ei[1], jnp.full((E_PAD - e,), N, jnp.int32)])
    row = row.reshape(E_PAD // 128, 128)
    col = col.reshape(E_PAD // 128, 128)
    zeros128 = jnp.zeros((RPW, 128), jnp.float32)
    ones_b = jnp.ones((128, 128), jnp.float32)

    degp = _sc_degree(col, zeros128, ones_b)
    dinv16, hs0 = _tc_prescale(degp, x)

    p1 = _sc_aggregate(hs0, row, col, zeros128)
    h1, st1 = _tc_agg_matmul(p1, hs0, dinv16, W1, b1.reshape(1, -1))
    hs1 = _tc_norm_prescale(h1, st1, ln1_w.reshape(1, -1),
                            ln1_b.reshape(1, -1), dinv16)

    p2 = _sc_aggregate(hs1, row, col, zeros128)
    h2, st2 = _tc_agg_matmul(p2, hs1, dinv16, W2, b2.reshape(1, -1))
    ts = _tc_norm_matmul3(h2, st2, ln2_w.reshape(1, -1),
                          ln2_b.reshape(1, -1), dinv16,
                          jnp.tile(W3, (1, 128)))

    p3 = _sc_aggregate(ts.reshape(1, N, 128), row, col, zeros128)
    out128 = _tc_final(p3, ts, dinv16,
                       jnp.broadcast_to(b3.reshape(1, 1), (1, 128)),
                       jnp.broadcast_to(ln3_w.reshape(1, 1), (1, 128)),
                       jnp.broadcast_to(ln3_b.reshape(1, 1), (1, 128)))
    return out128[:, 0]
